# Initial kernel scaffold; baseline (speedup 1.0000x reference)
import jax
import jax.numpy as jnp
from jax.experimental import pallas as pl


def kernel(node_features, edge_index, edge_norm, edge_type, bases, comp, root_w, root_b, gc_w_self, gc_w_nbr, gc_b):
    raise NotImplementedError("write your pallas kernel here")


# speedup vs baseline: 8.4720x; 8.4720x over previous
import functools

import jax
import jax.numpy as jnp
from jax import lax
from jax.experimental import pallas as pl
from jax.experimental.pallas import tpu as pltpu
from jax.experimental.pallas import tpu_sc as plsc

N = 10000
E = 320000
GD = 200
H = 100
R = 8
DP = 128
NC = 2
NS = 16
NW = NC * NS
CH = 80
EPT = E // NW
NCH = EPT // CH
EPT2 = E // NS
NCH2 = EPT2 // CH
NK = N * R
NKP = 80128
KSL = NKP // NS
NP = 10112
NSLP = NP // NS
L = 16


def _sc_mesh():
    return plsc.VectorSubcoreMesh(core_axis_name="c", subcore_axis_name="s")


def _splat(vec, lane):
    idx = jnp.full((L, 1), lane, jnp.int32)
    return lax.gather(
        vec, idx,
        dimension_numbers=lax.GatherDimensionNumbers(
            offset_dims=(), collapsed_slice_dims=(0,), start_index_map=(0,)),
        slice_sizes=(1,),
        mode=lax.GatherScatterMode.PROMISE_IN_BOUNDS)


def _k2_body(dst_hbm, typ_hbm, w_hbm, cnt_sh, dstv, typv, keys, onesv,
             slicebuf, wv):
    cid = lax.axis_index("c")
    sid = lax.axis_index("s")

    @pl.when(cid == 0)
    def _():
        def zb(i, _):
            slicebuf[pl.ds(i * L, L)] = jnp.zeros((L,), jnp.float32)
            return _
        lax.fori_loop(0, KSL // L, zb, None)
        pltpu.sync_copy(slicebuf, cnt_sh.at[pl.ds(sid * KSL, KSL)])
        for i in range(CH // L):
            onesv[pl.ds(i * L, L)] = jnp.ones((L,), jnp.float32)
        plsc.subcore_barrier()

        ebase = sid * EPT2

        def hchunk(j, _):
            e0 = ebase + j * CH
            pltpu.sync_copy(dst_hbm.at[pl.ds(e0, CH)], dstv)
            pltpu.sync_copy(typ_hbm.at[pl.ds(e0, CH)], typv)
            for i in range(CH // L):
                d = dstv[pl.ds(i * L, L)]
                t = typv[pl.ds(i * L, L)]
                keys[j, pl.ds(i * L, L)] = t * N + d
            pltpu.sync_copy(onesv, cnt_sh.at[keys.at[j]], add=True)
            return _
        lax.fori_loop(0, NCH2, hchunk, None)
        plsc.subcore_barrier()

        pltpu.sync_copy(cnt_sh.at[pl.ds(sid * KSL, KSL)], slicebuf)

        def inv(i, _):
            v = slicebuf[pl.ds(i * L, L)]
            slicebuf[pl.ds(i * L, L)] = 1.0 / jnp.maximum(v, 1.0)
            return _
        lax.fori_loop(0, KSL // L, inv, None)
        pltpu.sync_copy(slicebuf, cnt_sh.at[pl.ds(sid * KSL, KSL)])
        plsc.subcore_barrier()

        def wchunk(j, _):
            pltpu.sync_copy(cnt_sh.at[keys.at[j]], wv)
            pltpu.sync_copy(wv, w_hbm.at[pl.ds(sid * EPT2 + j * CH, CH)])
            return _
        lax.fori_loop(0, NCH2, wchunk, None)


def _k2(dst, typ):
    body = functools.partial(
        pl.kernel,
        out_type=jax.ShapeDtypeStruct((E,), jnp.float32),
        mesh=_sc_mesh(),
        scratch_types=[
            pltpu.VMEM_SHARED((NKP,), jnp.float32),
            pltpu.VMEM((CH,), jnp.int32),
            pltpu.VMEM((CH,), jnp.int32),
            pltpu.VMEM((NCH2, CH), jnp.int32),
            pltpu.VMEM((CH,), jnp.float32),
            pltpu.VMEM((KSL,), jnp.float32),
            pltpu.VMEM((CH,), jnp.float32),
        ],
    )(_k2_body)
    return body(dst, typ)


def _init_and_barrier(acc_sh, stage, sid):
    for i in range(CH):
        for jj in range(DP // L):
            stage[i, pl.ds(jj * L, L)] = jnp.zeros((L,), jnp.float32)
    for k in range(NSLP // CH + (1 if NSLP % CH else 0)):
        r0 = k * CH
        rn = min(CH, NSLP - r0)
        pltpu.sync_copy(stage.at[pl.ds(0, rn)],
                        acc_sh.at[pl.ds(sid * NSLP + r0, rn)])
    plsc.subcore_barrier()


def _flush_partial(acc_sh, stage, out_hbm, cid, sid):
    for k in range(NSLP // CH + (1 if NSLP % CH else 0)):
        r0 = k * CH
        rn = min(CH, NSLP - r0)
        pltpu.sync_copy(acc_sh.at[pl.ds(sid * NSLP + r0, rn)],
                        stage.at[pl.ds(0, rn)])
        pltpu.sync_copy(stage.at[pl.ds(0, rn)],
                        out_hbm.at[cid, pl.ds(sid * NSLP + r0, rn)])


def _k3_body(table_hbm, src_hbm, typ_hbm, dst_hbm, w_hbm, aggp_hbm,
             agg_sh, srcv, typv, skey, dkey, rows, wv, stage):
    cid = lax.axis_index("c")
    sid = lax.axis_index("s")
    wid = sid * NC + cid

    _init_and_barrier(agg_sh, stage, sid)

    ebase = wid * EPT

    def chunk(j, _):
        e0 = ebase + j * CH
        pltpu.sync_copy(src_hbm.at[pl.ds(e0, CH)], srcv)
        pltpu.sync_copy(typ_hbm.at[pl.ds(e0, CH)], typv)
        pltpu.sync_copy(dst_hbm.at[pl.ds(e0, CH)], dkey.at[0])
        pltpu.sync_copy(w_hbm.at[pl.ds(e0, CH)], wv)
        for i in range(CH // L):
            s = srcv[pl.ds(i * L, L)]
            t = typv[pl.ds(i * L, L)]
            skey[0, pl.ds(i * L, L)] = t * N + s
        pltpu.sync_copy(table_hbm.at[skey.at[0]], rows)

        for io in range(CH // L):
            wvec = wv[pl.ds(io * L, L)]
            for il in range(L):
                ws = _splat(wvec, il)
                i = io * L + il
                for jj in range(DP // L):
                    rows[i, pl.ds(jj * L, L)] = rows[i, pl.ds(jj * L, L)] * ws
        pltpu.sync_copy(rows, agg_sh.at[dkey.at[0]], add=True)
        return _
    lax.fori_loop(0, NCH, chunk, None)
    plsc.subcore_barrier()
    _flush_partial(agg_sh, stage, aggp_hbm, cid, sid)


def _k3(table, src, typ, dst, w):
    body = functools.partial(
        pl.kernel,
        out_type=jax.ShapeDtypeStruct((NC, NP, DP), jnp.float32),
        mesh=_sc_mesh(),
        scratch_types=[
            pltpu.VMEM_SHARED((NP, DP), jnp.float32),
            pltpu.VMEM((CH,), jnp.int32),
            pltpu.VMEM((CH,), jnp.int32),
            pltpu.VMEM((1, CH), jnp.int32),
            pltpu.VMEM((1, CH), jnp.int32),
            pltpu.VMEM((CH, DP), jnp.float32),
            pltpu.VMEM((CH,), jnp.float32),
            pltpu.VMEM((CH, DP), jnp.float32),
        ],
    )(_k3_body)
    return body(table, src, typ, dst, w)


def _k5_body(h1_hbm, src_hbm, dst_hbm, nbrp_hbm,
             acc_sh, skey, dkey, rows, stage):
    cid = lax.axis_index("c")
    sid = lax.axis_index("s")
    wid = sid * NC + cid

    _init_and_barrier(acc_sh, stage, sid)

    ebase = wid * EPT

    def chunk(j, _):
        e0 = ebase + j * CH
        pltpu.sync_copy(src_hbm.at[pl.ds(e0, CH)], skey.at[0])
        pltpu.sync_copy(dst_hbm.at[pl.ds(e0, CH)], dkey.at[0])
        pltpu.sync_copy(h1_hbm.at[skey.at[0]], rows)
        pltpu.sync_copy(rows, acc_sh.at[dkey.at[0]], add=True)
        return _
    lax.fori_loop(0, NCH, chunk, None)
    plsc.subcore_barrier()
    _flush_partial(acc_sh, stage, nbrp_hbm, cid, sid)


def _k5(h1, src, dst):
    body = functools.partial(
        pl.kernel,
        out_type=jax.ShapeDtypeStruct((NC, NP, DP), jnp.float32),
        mesh=_sc_mesh(),
        scratch_types=[
            pltpu.VMEM_SHARED((NP, DP), jnp.float32),
            pltpu.VMEM((1, CH), jnp.int32),
            pltpu.VMEM((1, CH), jnp.int32),
            pltpu.VMEM((CH, DP), jnp.float32),
            pltpu.VMEM((CH, DP), jnp.float32),
        ],
    )(_k5_body)
    return body(h1, src, dst)


BN = 400


def _k1_tc(x_ref, w_ref, b_ref, out_ref):
    acc = jnp.dot(x_ref[...], w_ref[0], preferred_element_type=jnp.float32)
    acc = acc + b_ref[0, 0]
    out_ref[0] = jnp.concatenate(
        [acc, jnp.zeros((BN, DP - H), jnp.float32)], axis=1)


def _k1(nf, w9, b9):
    return pl.pallas_call(
        _k1_tc,
        grid=(R + 1, N // BN),
        in_specs=[
            pl.BlockSpec((BN, GD), lambda r, i: (i, 0)),
            pl.BlockSpec((1, GD, H), lambda r, i: (r, 0, 0)),
            pl.BlockSpec((1, 1, H), lambda r, i: (r, 0, 0)),
        ],
        out_specs=pl.BlockSpec((1, BN, DP), lambda r, i: (r, i, 0)),
        out_shape=jax.ShapeDtypeStruct((R + 1, N, DP), jnp.float32),
    )(nf, w9, b9)


def _k0_tc(comp_ref, bases_ref, out_ref):
    out_ref[...] = jnp.dot(comp_ref[...], bases_ref[...],
                           preferred_element_type=jnp.float32)


def _k0(comp, bases2):
    nb = comp.shape[1]
    return pl.pallas_call(
        _k0_tc,
        out_shape=jax.ShapeDtypeStruct((R, GD * H), jnp.float32),
    )(comp, bases2)


def _k4_tc(aggp_ref, xroot_ref, out_ref):
    out_ref[...] = aggp_ref[0] + aggp_ref[1] + xroot_ref[...]


def _k4(aggp, xroot):
    return pl.pallas_call(
        _k4_tc,
        grid=(N // BN,),
        in_specs=[
            pl.BlockSpec((NC, BN, DP), lambda i: (0, i, 0)),
            pl.BlockSpec((BN, DP), lambda i: (i, 0)),
        ],
        out_specs=pl.BlockSpec((BN, DP), lambda i: (i, 0)),
        out_shape=jax.ShapeDtypeStruct((N, DP), jnp.float32),
    )(aggp, xroot)


def _k6_tc(h1_ref, nbrp_ref, ws_ref, wn_ref, b_ref, out_ref):
    nbr = nbrp_ref[0] + nbrp_ref[1]
    out_ref[...] = (
        jnp.dot(h1_ref[...], ws_ref[...], preferred_element_type=jnp.float32)
        + jnp.dot(nbr, wn_ref[...], preferred_element_type=jnp.float32)
        + b_ref[...])


def _k6(h1, nbrp, wsp, wnp, b):
    return pl.pallas_call(
        _k6_tc,
        grid=(N // BN,),
        in_specs=[
            pl.BlockSpec((BN, DP), lambda i: (i, 0)),
            pl.BlockSpec((NC, BN, DP), lambda i: (0, i, 0)),
            pl.BlockSpec((DP, H), lambda i: (0, 0)),
            pl.BlockSpec((DP, H), lambda i: (0, 0)),
            pl.BlockSpec((1, H), lambda i: (0, 0)),
        ],
        out_specs=pl.BlockSpec((BN, H), lambda i: (i, 0)),
        out_shape=jax.ShapeDtypeStruct((N, H), jnp.float32),
    )(h1, nbrp, wsp, wnp, b)


@jax.jit
def kernel(node_features, edge_index, edge_norm, edge_type, bases, comp,
           root_w, root_b, gc_w_self, gc_w_nbr, gc_b):
    del edge_norm
    src = edge_index[0]
    dst = edge_index[1]

    w_flat = _k0(comp, bases.reshape(bases.shape[0], GD * H))
    w9 = jnp.concatenate(
        [w_flat.reshape(R, GD, H), root_w[None]], axis=0)
    b9 = jnp.concatenate(
        [jnp.zeros((R, H), jnp.float32), root_b[None]], axis=0)[:, None, :]

    table9 = _k1(node_features, w9, b9)
    table = table9[:R].reshape(R * N, DP)
    xroot = table9[R]

    w_edge = _k2(dst, edge_type)
    aggp = _k3(table, src, edge_type, dst, w_edge)
    h1 = _k4(aggp[:, :N], xroot)
    nbrp = _k5(h1, src, dst)[:, :N]

    wsp = jnp.concatenate(
        [gc_w_self, jnp.zeros((DP - H, H), jnp.float32)], axis=0)
    wnp = jnp.concatenate(
        [gc_w_nbr, jnp.zeros((DP - H, H), jnp.float32)], axis=0)
    return _k6(h1, nbrp, wsp, wnp, gc_b[None])
